# Initial kernel scaffold; baseline (speedup 1.0000x reference)
import jax
import jax.numpy as jnp
from jax.experimental import pallas as pl


def kernel(x, edge_index):
    raise NotImplementedError("write your pallas kernel here")


# speedup vs baseline: 4.7312x; 4.7312x over previous
import functools

import jax
import jax.numpy as jnp
from jax import lax
from jax.experimental import pallas as pl
from jax.experimental.pallas import tpu as pltpu
from jax.experimental.pallas import tpu_sc as plsc

N = 10000
E = 320000
D = 128
DP = 144
NC, NS = 2, 16
NW = NC * NS
K = 128
CHUNKS = 79
EPW = CHUNKS * K
EP = NW * EPW
NR = 10240
RPS = NR // NS
ZR = 16
PAD_DST = N + 8


def _neg_log_table(x):
    def body(x_ref, o_ref):
        o_ref[...] = -jnp.log(x_ref[...])
    return pl.pallas_call(
        body, out_shape=jax.ShapeDtypeStruct((N, D), jnp.float32))(x)


def _combine(x, part):
    def body(x_ref, p_ref, o_ref):
        s = p_ref[0] + p_ref[1]
        sv = s[0:N, :]
        s_feat = sv[:, 0:D]
        deg = sv[:, D:DP].sum(axis=1)
        num = (x_ref[...] * s_feat).sum(axis=1)
        o_ref[...] = jnp.where(deg > 0, num / deg, 0.0)[:, None]
    return pl.pallas_call(
        body, out_shape=jax.ShapeDtypeStruct((N, 1), jnp.float32))(x, part)


def _make_sc_scatter():
    mesh = plsc.VectorSubcoreMesh(core_axis_name="c", subcore_axis_name="s")

    @functools.partial(
        pl.kernel,
        out_type=jax.ShapeDtypeStruct((NC, NR, DP), jnp.float32),
        mesh=mesh,
        compiler_params=pltpu.CompilerParams(use_tc_tiling_on_sc=False),
        scratch_types=[
            pltpu.VMEM((K,), jnp.int32),
            pltpu.VMEM((K,), jnp.int32),
            pltpu.VMEM((K, DP), jnp.float32),
            pltpu.VMEM((ZR, DP), jnp.float32),
            pltpu.VMEM_SHARED((NR, DP), jnp.float32),
            pltpu.SemaphoreType.DMA,
        ],
    )
    def sc_scatter(lp_hbm, srcp_hbm, dstp_hbm, part_hbm,
                   src_c, dst_c, rows_v, zero_v, acc_sh, sem):
        cid = lax.axis_index("c")
        sid = lax.axis_index("s")
        wid = sid * NC + cid

        def zrow(r, carry):
            for c9 in range(DP // 16):
                zero_v[r, pl.ds(c9 * 16, 16)] = jnp.zeros((16,), jnp.float32)
            return carry
        lax.fori_loop(0, ZR, zrow, 0)

        def zcp(i, carry):
            pltpu.sync_copy(zero_v, acc_sh.at[pl.ds(sid * RPS + i * ZR, ZR)])
            return carry
        lax.fori_loop(0, RPS // ZR, zcp, 0)
        plsc.subcore_barrier()

        def step(c, carry):
            pltpu.sync_copy(srcp_hbm.at[wid, c], src_c)
            pltpu.sync_copy(dstp_hbm.at[wid, c], dst_c)
            pltpu.async_copy(lp_hbm.at[src_c], rows_v, sem).wait()
            pltpu.sync_copy(rows_v, acc_sh.at[dst_c], add=True)
            return carry
        lax.fori_loop(0, CHUNKS, step, 0)
        plsc.subcore_barrier()

        pltpu.sync_copy(acc_sh.at[pl.ds(sid * RPS, RPS)],
                        part_hbm.at[cid, pl.ds(sid * RPS, RPS)])

    return sc_scatter


_sc_scatter = _make_sc_scatter()


def kernel(x, edge_index):
    src = edge_index[0]
    dst = edge_index[1]
    srcp = jnp.pad(src, (0, EP - E)).reshape(NW, CHUNKS, K)
    dstp = jnp.pad(dst, (0, EP - E), constant_values=PAD_DST).reshape(
        NW, CHUNKS, K)

    logt = _neg_log_table(x)
    table = jnp.concatenate(
        [logt, jnp.ones((N, 1), jnp.float32), jnp.zeros((N, DP - D - 1),
                                                        jnp.float32)], axis=1)
    part = _sc_scatter(table, srcp, dstp)
    return _combine(x, part).reshape(N)
